# baseline (speedup 1.0000x reference)
import jax
import jax.numpy as jnp
from jax.experimental import pallas as pl

_ROWS = 512


def _make_kernel(sb):
    def _add_kernel(x_ref, pos_ref, o_ref):
        s = pl.program_id(0) % sb
        o_ref[...] = x_ref[...] + pos_ref[pl.ds(s * _ROWS, _ROWS), :]

    return _add_kernel


def kernel(x, pos_table):
    B, S, D = x.shape
    pos = pos_table[:S]
    x2 = x.reshape(B * S, D)
    sb = S // _ROWS
    out = pl.pallas_call(
        _make_kernel(sb),
        grid=(B * sb,),
        in_specs=[
            pl.BlockSpec((_ROWS, D), lambda i: (i, 0)),
            pl.BlockSpec((S, D), lambda i: (0, 0)),
        ],
        out_specs=pl.BlockSpec((_ROWS, D), lambda i: (i, 0)),
        out_shape=jax.ShapeDtypeStruct((B * S, D), x.dtype),
    )(x2, pos)
    return out.reshape(B, S, D)


# speedup vs baseline: 1.1542x; 1.1542x over previous
import jax
import jax.numpy as jnp
from jax.experimental import pallas as pl


def _add_kernel(x_ref, pos_ref, o_ref):
    o_ref[...] = x_ref[...] + pos_ref[...]


def kernel(x, pos_table):
    B, S, D = x.shape
    pos = pos_table[:S]
    x2 = x.reshape(B * S, D)
    out = pl.pallas_call(
        _add_kernel,
        grid=(B,),
        in_specs=[
            pl.BlockSpec((S, D), lambda b: (b, 0)),
            pl.BlockSpec((S, D), lambda b: (0, 0)),
        ],
        out_specs=pl.BlockSpec((S, D), lambda b: (b, 0)),
        out_shape=jax.ShapeDtypeStruct((B * S, D), x.dtype),
    )(x2, pos)
    return out.reshape(B, S, D)
